# Initial kernel scaffold; baseline (speedup 1.0000x reference)
import jax
import jax.numpy as jnp
from jax.experimental import pallas as pl


def kernel(postag_ids, bias_table):
    raise NotImplementedError("write your pallas kernel here")


# speedup vs baseline: 167.0161x; 167.0161x over previous
import jax
import jax.numpy as jnp
from jax.experimental import pallas as pl

_NT = 50


def _body(idsi_ref, idsj_ref, w_ref, out_ref):
    ti = out_ref.shape[2]
    ell = out_ref.shape[3]
    idsi = idsi_ref[0]
    idsj = idsj_ref[0]
    w = w_ref[0]

    s_iota = jax.lax.broadcasted_iota(jnp.int32, (_NT, ell), 0)
    oj = (idsj == s_iota).astype(jnp.float32)
    p = jnp.dot(w, oj, preferred_element_type=jnp.float32)

    t_iota = jax.lax.broadcasted_iota(jnp.int32, (_NT, ti), 0)
    oit = (idsi == t_iota).astype(jnp.float32)
    res = jax.lax.dot_general(
        oit, p, (((0,), (0,)), ((), ())),
        preferred_element_type=jnp.float32)
    out_ref[0, 0] = res


def kernel(postag_ids, bias_table):
    ids = postag_ids.astype(jnp.int32)
    b, ell = ids.shape
    nh = bias_table.shape[1]
    ti = 256
    ids3 = ids.reshape(b, 1, ell)
    w = bias_table.T.reshape(nh, _NT, _NT)

    grid = (b, nh, ell // ti)
    return pl.pallas_call(
        _body,
        grid=grid,
        in_specs=[
            pl.BlockSpec((1, 1, ti), lambda bb, hh, it: (bb, 0, it)),
            pl.BlockSpec((1, 1, ell), lambda bb, hh, it: (bb, 0, 0)),
            pl.BlockSpec((1, _NT, _NT), lambda bb, hh, it: (hh, 0, 0)),
        ],
        out_specs=pl.BlockSpec((1, 1, ti, ell),
                               lambda bb, hh, it: (bb, hh, it, 0)),
        out_shape=jax.ShapeDtypeStruct((b, nh, ell, ell), jnp.float32),
    )(ids3, ids3, w)


# speedup vs baseline: 230.2246x; 1.3785x over previous
import jax
import jax.numpy as jnp
from jax.experimental import pallas as pl

_NT = 50


def _body(idsi_ref, idsj_ref, w_ref, out_ref):
    ti = out_ref.shape[2]
    ell = out_ref.shape[3]
    idsi = idsi_ref[0]
    idsj = idsj_ref[0]
    w = w_ref[0]

    s_iota = jax.lax.broadcasted_iota(jnp.int32, (_NT, ell), 0)
    oj = (idsj == s_iota).astype(jnp.float32)
    p = jnp.dot(w, oj, preferred_element_type=jnp.float32)

    t_iota = jax.lax.broadcasted_iota(jnp.int32, (_NT, ti), 0)
    oit = (idsi == t_iota).astype(jnp.float32)
    res = jax.lax.dot_general(
        oit, p, (((0,), (0,)), ((), ())),
        preferred_element_type=jnp.float32)
    out_ref[0, 0] = res


def kernel(postag_ids, bias_table):
    ids = postag_ids.astype(jnp.int32)
    b, ell = ids.shape
    nh = bias_table.shape[1]
    ti = 512
    ids3 = ids.reshape(b, 1, ell)
    w = bias_table.T.reshape(nh, _NT, _NT)

    grid = (b, nh, ell // ti)
    return pl.pallas_call(
        _body,
        grid=grid,
        in_specs=[
            pl.BlockSpec((1, 1, ti), lambda bb, hh, it: (bb, 0, it)),
            pl.BlockSpec((1, 1, ell), lambda bb, hh, it: (bb, 0, 0)),
            pl.BlockSpec((1, _NT, _NT), lambda bb, hh, it: (hh, 0, 0)),
        ],
        out_specs=pl.BlockSpec((1, 1, ti, ell),
                               lambda bb, hh, it: (bb, hh, it, 0)),
        out_shape=jax.ShapeDtypeStruct((b, nh, ell, ell), jnp.float32),
    )(ids3, ids3, w)


# speedup vs baseline: 254.4740x; 1.1053x over previous
import jax
import jax.numpy as jnp
from jax.experimental import pallas as pl

_NT = 50


def _body(idsi_ref, idsj_ref, w_ref, out_ref):
    ti = out_ref.shape[2]
    ell = out_ref.shape[3]
    idsi = idsi_ref[0]
    idsj = idsj_ref[0]
    w = w_ref[0]

    s_iota = jax.lax.broadcasted_iota(jnp.int32, (_NT, ell), 0)
    oj = (idsj == s_iota).astype(jnp.float32)
    p = jnp.dot(w, oj, preferred_element_type=jnp.float32)

    t_iota = jax.lax.broadcasted_iota(jnp.int32, (_NT, ti), 0)
    oit = (idsi == t_iota).astype(jnp.float32)
    res = jax.lax.dot_general(
        oit, p, (((0,), (0,)), ((), ())),
        preferred_element_type=jnp.float32)
    out_ref[0, 0] = res


def kernel(postag_ids, bias_table):
    ids = postag_ids.astype(jnp.int32)
    b, ell = ids.shape
    nh = bias_table.shape[1]
    ti = 1024
    ids3 = ids.reshape(b, 1, ell)
    w = bias_table.T.reshape(nh, _NT, _NT)

    grid = (b, nh, ell // ti)
    return pl.pallas_call(
        _body,
        grid=grid,
        in_specs=[
            pl.BlockSpec((1, 1, ti), lambda bb, hh, it: (bb, 0, it)),
            pl.BlockSpec((1, 1, ell), lambda bb, hh, it: (bb, 0, 0)),
            pl.BlockSpec((1, _NT, _NT), lambda bb, hh, it: (hh, 0, 0)),
        ],
        out_specs=pl.BlockSpec((1, 1, ti, ell),
                               lambda bb, hh, it: (bb, hh, it, 0)),
        out_shape=jax.ShapeDtypeStruct((b, nh, ell, ell), jnp.float32),
    )(ids3, ids3, w)
